# baseline (device time: 110428 ns/iter reference)
import os

import jax
import jax.numpy as jnp
from jax import lax
from jax.experimental import pallas as pl
from jax.experimental.pallas import tpu as pltpu

N_DEV = 16
SQ = 1024
SKV_LOCAL = 1024
HQ = 8
DH = 128
BLK = 64
CHUNK = SQ // N_DEV
NGRP = 4
GROWS = SQ // NGRP
SCALE = 0.08838834764831843
_SKIP_COMM = bool(os.environ.get("SKIP_COMM"))


def _rem(v):
    return lax.rem(v + 2 * N_DEV, N_DEV)


def kernel(x, Wq, K_ext, V_ext, Wo):
    def body(x_ref, wq_ref, k_ref, v_ref, wo_ref, out_ref,
             q_ref, kg_ref, vg_ref, acc_ref, l_ref, racc_ref, rl_ref,
             rs_s_sems, rs_r_sems, rsl_s_sems, rsl_r_sems,
             ag_s_sems, ag_r_sems):
        my = lax.axis_index("i")
        left = _rem(my - 1)
        right = _rem(my + 1)

        xb = x_ref[0].astype(jnp.bfloat16)
        wqb = wq_ref[...].astype(jnp.bfloat16)
        q = lax.dot(xb, wqb, preferred_element_type=jnp.float32) * SCALE
        q_ref[...] = q.reshape(N_DEV, CHUNK, HQ * DH).astype(jnp.bfloat16)
        for g in range(NGRP):
            blks = [g + NGRP * b for b in range(NGRP)]
            for h in range(HQ):
                kg_ref[g, h] = jnp.concatenate(
                    [k_ref[0, c * BLK:(c + 1) * BLK, h, :] for c in blks],
                    axis=0).astype(jnp.bfloat16)
                vg_ref[g, h] = jnp.concatenate(
                    [v_ref[0, c * BLK:(c + 1) * BLK, h, :] for c in blks],
                    axis=0).astype(jnp.bfloat16)

        def compute_chunk(c):
            g = lax.rem(c, NGRP)
            for h in range(HQ):
                qch = q_ref[c, :, h * DH:(h + 1) * DH]
                s = lax.dot_general(qch, kg_ref[g, h], (((1,), (1,)), ((), ())),
                                    preferred_element_type=jnp.float32)
                w = jnp.exp(s)
                l_ref[c, h, :] = jnp.sum(w, axis=1)
                acc_ref[c, :, h, :] = lax.dot(
                    w.astype(jnp.bfloat16), vg_ref[g, h],
                    preferred_element_type=jnp.float32)

        compute_chunk(_rem(my - 8))
        compute_chunk(_rem(my + 7))
        compute_chunk(_rem(my - 7))
        compute_chunk(_rem(my + 6))

        barrier = pltpu.get_barrier_semaphore()
        pl.semaphore_signal(barrier, inc=1, device_id=(left,),
                            device_id_type=pl.DeviceIdType.MESH)
        pl.semaphore_signal(barrier, inc=1, device_id=(right,),
                            device_id_type=pl.DeviceIdType.MESH)
        pl.semaphore_wait(barrier, 2)

        for st in range(8 if not _SKIP_COMM else 0):
            cl_s = _rem(my - 8 + st)
            rd_l = pltpu.make_async_remote_copy(
                src_ref=acc_ref.at[cl_s], dst_ref=racc_ref.at[st],
                send_sem=rs_s_sems.at[st], recv_sem=rs_r_sems.at[st],
                device_id=(left,), device_id_type=pl.DeviceIdType.MESH)
            rdl_l = pltpu.make_async_remote_copy(
                src_ref=l_ref.at[cl_s], dst_ref=rl_ref.at[st],
                send_sem=rsl_s_sems.at[st], recv_sem=rsl_r_sems.at[st],
                device_id=(left,), device_id_type=pl.DeviceIdType.MESH)
            rd_l.start()
            rdl_l.start()
            if st < 7:
                cr_s = _rem(my + 7 - st)
                rd_r = pltpu.make_async_remote_copy(
                    src_ref=acc_ref.at[cr_s], dst_ref=racc_ref.at[8 + st],
                    send_sem=rs_s_sems.at[8 + st],
                    recv_sem=rs_r_sems.at[8 + st],
                    device_id=(right,), device_id_type=pl.DeviceIdType.MESH)
                rdl_r = pltpu.make_async_remote_copy(
                    src_ref=l_ref.at[cr_s], dst_ref=rl_ref.at[8 + st],
                    send_sem=rsl_s_sems.at[8 + st],
                    recv_sem=rsl_r_sems.at[8 + st],
                    device_id=(right,), device_id_type=pl.DeviceIdType.MESH)
                rd_r.start()
                rdl_r.start()
            if st < 6:
                compute_chunk(_rem(my - 6 + st))
                compute_chunk(_rem(my + 5 - st))
            rd_l.wait()
            rdl_l.wait()
            cl = _rem(my - 7 + st)
            acc_ref[cl] = acc_ref[cl] + racc_ref[st]
            l_ref[cl] = l_ref[cl] + rl_ref[st]
            if st < 7:
                rd_r.wait()
                rdl_r.wait()
                cr = _rem(my + 6 - st)
                acc_ref[cr] = acc_ref[cr] + racc_ref[8 + st]
                l_ref[cr] = l_ref[cr] + rl_ref[8 + st]

        accc = acc_ref[my]
        lc = l_ref[my]
        parts = []
        for h in range(HQ):
            parts.append(accc[:, h, :] / lc[h][:, None])
        ctx = jnp.concatenate(parts, axis=1).astype(jnp.bfloat16)
        wob = wo_ref[...].astype(jnp.bfloat16)
        outc = lax.dot(ctx, wob,
                       preferred_element_type=jnp.float32).astype(jnp.bfloat16)
        out_ref[0, pl.ds(my * CHUNK, CHUNK), :] = outc

        for t in range(8 if not _SKIP_COMM else 0):
            g_r = _rem(my - t)
            ag_r = pltpu.make_async_remote_copy(
                src_ref=out_ref.at[0, pl.ds(g_r * CHUNK, CHUNK), :],
                dst_ref=out_ref.at[0, pl.ds(g_r * CHUNK, CHUNK), :],
                send_sem=ag_s_sems.at[t], recv_sem=ag_r_sems.at[t],
                device_id=(right,), device_id_type=pl.DeviceIdType.MESH)
            ag_r.start()
            if t < 7:
                g_l = _rem(my + t)
                ag_l = pltpu.make_async_remote_copy(
                    src_ref=out_ref.at[0, pl.ds(g_l * CHUNK, CHUNK), :],
                    dst_ref=out_ref.at[0, pl.ds(g_l * CHUNK, CHUNK), :],
                    send_sem=ag_s_sems.at[8 + t], recv_sem=ag_r_sems.at[8 + t],
                    device_id=(left,), device_id_type=pl.DeviceIdType.MESH)
                ag_l.start()
            ag_r.wait()
            if t < 7:
                ag_l.wait()

    return pl.pallas_call(
        body,
        out_shape=jax.ShapeDtypeStruct((1, SQ, HQ * DH), jnp.bfloat16),
        in_specs=[pl.BlockSpec(memory_space=pltpu.VMEM)] * 5,
        out_specs=pl.BlockSpec(memory_space=pltpu.VMEM),
        scratch_shapes=[
            pltpu.VMEM((N_DEV, CHUNK, HQ * DH), jnp.bfloat16),
            pltpu.VMEM((NGRP, HQ, GROWS, DH), jnp.bfloat16),
            pltpu.VMEM((NGRP, HQ, GROWS, DH), jnp.bfloat16),
            pltpu.VMEM((N_DEV, CHUNK, HQ, DH), jnp.float32),
            pltpu.VMEM((N_DEV, HQ, CHUNK), jnp.float32),
            pltpu.VMEM((N_DEV - 1, CHUNK, HQ, DH), jnp.float32),
            pltpu.VMEM((N_DEV - 1, HQ, CHUNK), jnp.float32),
            pltpu.SemaphoreType.DMA((N_DEV - 1,)),
            pltpu.SemaphoreType.DMA((N_DEV - 1,)),
            pltpu.SemaphoreType.DMA((N_DEV - 1,)),
            pltpu.SemaphoreType.DMA((N_DEV - 1,)),
            pltpu.SemaphoreType.DMA((N_DEV - 1,)),
            pltpu.SemaphoreType.DMA((N_DEV - 1,)),
        ],
        compiler_params=pltpu.CompilerParams(
            collective_id=0,
            vmem_limit_bytes=120 * 1024 * 1024,
        ),
    )(x, Wq, K_ext, V_ext, Wo)


# device time: 108144 ns/iter; 1.0211x vs baseline; 1.0211x over previous
import os

import jax
import jax.numpy as jnp
from jax import lax
from jax.experimental import pallas as pl
from jax.experimental.pallas import tpu as pltpu

N_DEV = 16
SQ = 1024
SKV_LOCAL = 1024
HQ = 8
DH = 128
BLK = 64
CHUNK = SQ // N_DEV
NGRP = 4
GROWS = SQ // NGRP
SCALE = 0.08838834764831843
_SKIP_COMM = bool(os.environ.get("SKIP_COMM"))


def _rem(v):
    return lax.rem(v + 2 * N_DEV, N_DEV)


def kernel(x, Wq, K_ext, V_ext, Wo):
    def body(x_ref, wq_ref, k_ref, v_ref, wo_ref, out_ref,
             q_ref, kg_ref, vg_ref, acc_ref, l_ref, racc_ref, rl_ref,
             rs_s_sems, rs_r_sems, rsl_s_sems, rsl_r_sems,
             ag_s_sems, ag_r_sems):
        my = lax.axis_index("i")
        left = _rem(my - 1)
        right = _rem(my + 1)

        xb = x_ref[0].astype(jnp.bfloat16)
        wqb = wq_ref[...].astype(jnp.bfloat16)
        q = lax.dot(xb, wqb, preferred_element_type=jnp.float32) * SCALE
        q_ref[...] = q.reshape(N_DEV, CHUNK, HQ * DH).astype(jnp.bfloat16)
        for g in range(NGRP):
            blks = [g + NGRP * b for b in range(NGRP)]
            for h in range(HQ):
                kg_ref[g, h] = jnp.concatenate(
                    [k_ref[0, c * BLK:(c + 1) * BLK, h, :] for c in blks],
                    axis=0).astype(jnp.bfloat16)
                vg_ref[g, h] = jnp.concatenate(
                    [v_ref[0, c * BLK:(c + 1) * BLK, h, :] for c in blks],
                    axis=0).astype(jnp.bfloat16)

        def compute_pair(c):
            c2 = _rem(c + 4)
            g = lax.rem(c, NGRP)
            qp = jnp.concatenate([q_ref[c], q_ref[c2]], axis=0)
            for h in range(HQ):
                qh = qp[:, h * DH:(h + 1) * DH]
                s = lax.dot_general(qh, kg_ref[g, h], (((1,), (1,)), ((), ())),
                                    preferred_element_type=jnp.float32)
                w = jnp.exp(s)
                ls = jnp.sum(w, axis=1)
                a = lax.dot(w.astype(jnp.bfloat16), vg_ref[g, h],
                            preferred_element_type=jnp.float32)
                acc_ref[c, :, h, :] = a[:CHUNK]
                acc_ref[c2, :, h, :] = a[CHUNK:]
                l_ref[c, h, :] = ls[:CHUNK]
                l_ref[c2, h, :] = ls[CHUNK:]

        compute_pair(_rem(my - 8))
        compute_pair(_rem(my + 7))
        compute_pair(_rem(my - 7))
        compute_pair(_rem(my + 6))

        barrier = pltpu.get_barrier_semaphore()
        pl.semaphore_signal(barrier, inc=1, device_id=(left,),
                            device_id_type=pl.DeviceIdType.MESH)
        pl.semaphore_signal(barrier, inc=1, device_id=(right,),
                            device_id_type=pl.DeviceIdType.MESH)
        pl.semaphore_wait(barrier, 2)

        for st in range(8):
            if not _SKIP_COMM:
                cl_s = _rem(my - 8 + st)
                rd_l = pltpu.make_async_remote_copy(
                    src_ref=acc_ref.at[cl_s], dst_ref=racc_ref.at[st],
                    send_sem=rs_s_sems.at[st], recv_sem=rs_r_sems.at[st],
                    device_id=(left,), device_id_type=pl.DeviceIdType.MESH)
                rdl_l = pltpu.make_async_remote_copy(
                    src_ref=l_ref.at[cl_s], dst_ref=rl_ref.at[st],
                    send_sem=rsl_s_sems.at[st], recv_sem=rsl_r_sems.at[st],
                    device_id=(left,), device_id_type=pl.DeviceIdType.MESH)
                rd_l.start()
                rdl_l.start()
                if st < 7:
                    cr_s = _rem(my + 7 - st)
                    rd_r = pltpu.make_async_remote_copy(
                        src_ref=acc_ref.at[cr_s], dst_ref=racc_ref.at[8 + st],
                        send_sem=rs_s_sems.at[8 + st],
                        recv_sem=rs_r_sems.at[8 + st],
                        device_id=(right,),
                        device_id_type=pl.DeviceIdType.MESH)
                    rdl_r = pltpu.make_async_remote_copy(
                        src_ref=l_ref.at[cr_s], dst_ref=rl_ref.at[8 + st],
                        send_sem=rsl_s_sems.at[8 + st],
                        recv_sem=rsl_r_sems.at[8 + st],
                        device_id=(right,),
                        device_id_type=pl.DeviceIdType.MESH)
                    rd_r.start()
                    rdl_r.start()
            if st == 0:
                compute_pair(_rem(my + 1))
                compute_pair(my)
            elif st == 1:
                compute_pair(_rem(my - 1))
                compute_pair(_rem(my - 2))
            if not _SKIP_COMM:
                rd_l.wait()
                rdl_l.wait()
                cl = _rem(my - 7 + st)
                acc_ref[cl] = acc_ref[cl] + racc_ref[st]
                l_ref[cl] = l_ref[cl] + rl_ref[st]
                if st < 7:
                    rd_r.wait()
                    rdl_r.wait()
                    cr = _rem(my + 6 - st)
                    acc_ref[cr] = acc_ref[cr] + racc_ref[8 + st]
                    l_ref[cr] = l_ref[cr] + rl_ref[8 + st]

        accc = acc_ref[my]
        lc = l_ref[my]
        parts = []
        for h in range(HQ):
            parts.append(accc[:, h, :] / lc[h][:, None])
        ctx = jnp.concatenate(parts, axis=1).astype(jnp.bfloat16)
        wob = wo_ref[...].astype(jnp.bfloat16)
        outc = lax.dot(ctx, wob,
                       preferred_element_type=jnp.float32).astype(jnp.bfloat16)
        out_ref[0, pl.ds(my * CHUNK, CHUNK), :] = outc

        for t in range(8 if not _SKIP_COMM else 0):
            g_r = _rem(my - t)
            ag_r = pltpu.make_async_remote_copy(
                src_ref=out_ref.at[0, pl.ds(g_r * CHUNK, CHUNK), :],
                dst_ref=out_ref.at[0, pl.ds(g_r * CHUNK, CHUNK), :],
                send_sem=ag_s_sems.at[t], recv_sem=ag_r_sems.at[t],
                device_id=(right,), device_id_type=pl.DeviceIdType.MESH)
            ag_r.start()
            if t < 7:
                g_l = _rem(my + t)
                ag_l = pltpu.make_async_remote_copy(
                    src_ref=out_ref.at[0, pl.ds(g_l * CHUNK, CHUNK), :],
                    dst_ref=out_ref.at[0, pl.ds(g_l * CHUNK, CHUNK), :],
                    send_sem=ag_s_sems.at[8 + t], recv_sem=ag_r_sems.at[8 + t],
                    device_id=(left,), device_id_type=pl.DeviceIdType.MESH)
                ag_l.start()
            ag_r.wait()
            if t < 7:
                ag_l.wait()

    return pl.pallas_call(
        body,
        out_shape=jax.ShapeDtypeStruct((1, SQ, HQ * DH), jnp.bfloat16),
        in_specs=[pl.BlockSpec(memory_space=pltpu.VMEM)] * 5,
        out_specs=pl.BlockSpec(memory_space=pltpu.VMEM),
        scratch_shapes=[
            pltpu.VMEM((N_DEV, CHUNK, HQ * DH), jnp.bfloat16),
            pltpu.VMEM((NGRP, HQ, GROWS, DH), jnp.bfloat16),
            pltpu.VMEM((NGRP, HQ, GROWS, DH), jnp.bfloat16),
            pltpu.VMEM((N_DEV, CHUNK, HQ, DH), jnp.float32),
            pltpu.VMEM((N_DEV, HQ, CHUNK), jnp.float32),
            pltpu.VMEM((N_DEV - 1, CHUNK, HQ, DH), jnp.float32),
            pltpu.VMEM((N_DEV - 1, HQ, CHUNK), jnp.float32),
            pltpu.SemaphoreType.DMA((N_DEV - 1,)),
            pltpu.SemaphoreType.DMA((N_DEV - 1,)),
            pltpu.SemaphoreType.DMA((N_DEV - 1,)),
            pltpu.SemaphoreType.DMA((N_DEV - 1,)),
            pltpu.SemaphoreType.DMA((N_DEV - 1,)),
            pltpu.SemaphoreType.DMA((N_DEV - 1,)),
        ],
        compiler_params=pltpu.CompilerParams(
            collective_id=0,
            vmem_limit_bytes=120 * 1024 * 1024,
        ),
    )(x, Wq, K_ext, V_ext, Wo)


# device time: 90965 ns/iter; 1.2140x vs baseline; 1.1889x over previous
import os

import jax
import jax.numpy as jnp
from jax import lax
from jax.experimental import pallas as pl
from jax.experimental.pallas import tpu as pltpu

N_DEV = 16
SQ = 1024
SKV_LOCAL = 1024
HQ = 8
DH = 128
BLK = 64
CHUNK = SQ // N_DEV
NGRP = 4
GROWS = SQ // NGRP
SCALE = 0.08838834764831843
_SKIP_COMM = bool(os.environ.get("SKIP_COMM"))
_SKIP_COMPUTE = bool(os.environ.get("SKIP_COMPUTE"))

RING = [0, 1, 2, 3, 7, 6, 5, 9, 10, 11, 15, 14, 13, 12, 8, 4]
PERM = [0] * N_DEV
for _p, _d in enumerate(RING):
    PERM[_d] = _p


def _rem(v):
    return lax.rem(v + 2 * N_DEV, N_DEV)


def kernel(x, Wq, K_ext, V_ext, Wo):
    ridx = lax.axis_index("i")
    rp_ = jnp.take(jnp.array(PERM, jnp.int32), ridx)
    lneigh = jnp.take(jnp.array(RING, jnp.int32), (rp_ - 1) % N_DEV)
    rneigh = jnp.take(jnp.array(RING, jnp.int32), (rp_ + 1) % N_DEV)
    idx = jnp.stack([rp_, lneigh, rneigh]).astype(jnp.int32)

    def body(idx_ref, x_ref, wq_ref, k_ref, v_ref, wo_ref, out_ref,
             q_ref, kg_ref, vg_ref, acc_ref, l_ref, racc_ref, rl_ref,
             rs_s_sems, rs_r_sems, rsl_s_sems, rsl_r_sems,
             ag_s_sems, ag_r_sems):
        rp = idx_ref[0]
        left = idx_ref[1]
        right = idx_ref[2]

        xb = x_ref[0].astype(jnp.bfloat16)
        wqb = wq_ref[...].astype(jnp.bfloat16)
        q = lax.dot(xb, wqb, preferred_element_type=jnp.float32) * SCALE
        q_ref[...] = q.reshape(N_DEV, CHUNK, HQ * DH).astype(jnp.bfloat16)
        for g in range(NGRP if not _SKIP_COMPUTE else 0):
            blks = [g + NGRP * b for b in range(NGRP)]
            for h in range(HQ):
                kg_ref[g, h] = jnp.concatenate(
                    [k_ref[0, c * BLK:(c + 1) * BLK, h, :] for c in blks],
                    axis=0).astype(jnp.bfloat16)
                vg_ref[g, h] = jnp.concatenate(
                    [v_ref[0, c * BLK:(c + 1) * BLK, h, :] for c in blks],
                    axis=0).astype(jnp.bfloat16)

        def compute_pair(c):
            if _SKIP_COMPUTE:
                return
            c2 = _rem(c + 4)
            g = lax.rem(c, NGRP)
            qp = jnp.concatenate([q_ref[c], q_ref[c2]], axis=0)
            for h in range(HQ):
                qh = qp[:, h * DH:(h + 1) * DH]
                s = lax.dot_general(qh, kg_ref[g, h], (((1,), (1,)), ((), ())),
                                    preferred_element_type=jnp.float32)
                w = jnp.exp(s)
                ls = jnp.sum(w, axis=1)
                a = lax.dot(w.astype(jnp.bfloat16), vg_ref[g, h],
                            preferred_element_type=jnp.float32)
                acc_ref[c, :, h, :] = a[:CHUNK]
                acc_ref[c2, :, h, :] = a[CHUNK:]
                l_ref[c, h, :] = ls[:CHUNK]
                l_ref[c2, h, :] = ls[CHUNK:]

        compute_pair(_rem(rp - 8))
        compute_pair(_rem(rp + 7))
        compute_pair(_rem(rp - 7))
        compute_pair(_rem(rp + 6))

        barrier = pltpu.get_barrier_semaphore()
        pl.semaphore_signal(barrier, inc=1, device_id=(left,),
                            device_id_type=pl.DeviceIdType.MESH)
        pl.semaphore_signal(barrier, inc=1, device_id=(right,),
                            device_id_type=pl.DeviceIdType.MESH)
        pl.semaphore_wait(barrier, 2)

        for st in range(8):
            if not _SKIP_COMM:
                cl_s = _rem(rp - 8 + st)
                rd_l = pltpu.make_async_remote_copy(
                    src_ref=acc_ref.at[cl_s], dst_ref=racc_ref.at[st],
                    send_sem=rs_s_sems.at[st], recv_sem=rs_r_sems.at[st],
                    device_id=(left,), device_id_type=pl.DeviceIdType.MESH)
                rdl_l = pltpu.make_async_remote_copy(
                    src_ref=l_ref.at[cl_s], dst_ref=rl_ref.at[st],
                    send_sem=rsl_s_sems.at[st], recv_sem=rsl_r_sems.at[st],
                    device_id=(left,), device_id_type=pl.DeviceIdType.MESH)
                rd_l.start()
                rdl_l.start()
                if st < 7:
                    cr_s = _rem(rp + 7 - st)
                    rd_r = pltpu.make_async_remote_copy(
                        src_ref=acc_ref.at[cr_s], dst_ref=racc_ref.at[8 + st],
                        send_sem=rs_s_sems.at[8 + st],
                        recv_sem=rs_r_sems.at[8 + st],
                        device_id=(right,),
                        device_id_type=pl.DeviceIdType.MESH)
                    rdl_r = pltpu.make_async_remote_copy(
                        src_ref=l_ref.at[cr_s], dst_ref=rl_ref.at[8 + st],
                        send_sem=rsl_s_sems.at[8 + st],
                        recv_sem=rsl_r_sems.at[8 + st],
                        device_id=(right,),
                        device_id_type=pl.DeviceIdType.MESH)
                    rd_r.start()
                    rdl_r.start()
            if st == 0:
                compute_pair(_rem(rp + 1))
                compute_pair(rp)
            elif st == 1:
                compute_pair(_rem(rp - 1))
                compute_pair(_rem(rp - 2))
            if not _SKIP_COMM:
                rd_l.wait()
                rdl_l.wait()
                cl = _rem(rp - 7 + st)
                acc_ref[cl] = acc_ref[cl] + racc_ref[st]
                l_ref[cl] = l_ref[cl] + rl_ref[st]
                if st < 7:
                    rd_r.wait()
                    rdl_r.wait()
                    cr = _rem(rp + 6 - st)
                    acc_ref[cr] = acc_ref[cr] + racc_ref[8 + st]
                    l_ref[cr] = l_ref[cr] + rl_ref[8 + st]

        accc = acc_ref[rp]
        lc = l_ref[rp]
        parts = []
        for h in range(HQ):
            parts.append(accc[:, h, :] / lc[h][:, None])
        ctx = jnp.concatenate(parts, axis=1).astype(jnp.bfloat16)
        wob = wo_ref[...].astype(jnp.bfloat16)
        outc = lax.dot(ctx, wob,
                       preferred_element_type=jnp.float32).astype(jnp.bfloat16)
        out_ref[0, pl.ds(rp * CHUNK, CHUNK), :] = outc

        for t in range(8 if not _SKIP_COMM else 0):
            g_r = _rem(rp - t)
            ag_r = pltpu.make_async_remote_copy(
                src_ref=out_ref.at[0, pl.ds(g_r * CHUNK, CHUNK), :],
                dst_ref=out_ref.at[0, pl.ds(g_r * CHUNK, CHUNK), :],
                send_sem=ag_s_sems.at[t], recv_sem=ag_r_sems.at[t],
                device_id=(right,), device_id_type=pl.DeviceIdType.MESH)
            ag_r.start()
            if t < 7:
                g_l = _rem(rp + t)
                ag_l = pltpu.make_async_remote_copy(
                    src_ref=out_ref.at[0, pl.ds(g_l * CHUNK, CHUNK), :],
                    dst_ref=out_ref.at[0, pl.ds(g_l * CHUNK, CHUNK), :],
                    send_sem=ag_s_sems.at[8 + t], recv_sem=ag_r_sems.at[8 + t],
                    device_id=(left,), device_id_type=pl.DeviceIdType.MESH)
                ag_l.start()
            ag_r.wait()
            if t < 7:
                ag_l.wait()

    return pl.pallas_call(
        body,
        out_shape=jax.ShapeDtypeStruct((1, SQ, HQ * DH), jnp.bfloat16),
        in_specs=[pl.BlockSpec(memory_space=pltpu.SMEM)]
        + [pl.BlockSpec(memory_space=pltpu.VMEM)] * 5,
        out_specs=pl.BlockSpec(memory_space=pltpu.VMEM),
        scratch_shapes=[
            pltpu.VMEM((N_DEV, CHUNK, HQ * DH), jnp.bfloat16),
            pltpu.VMEM((NGRP, HQ, GROWS, DH), jnp.bfloat16),
            pltpu.VMEM((NGRP, HQ, GROWS, DH), jnp.bfloat16),
            pltpu.VMEM((N_DEV, CHUNK, HQ, DH), jnp.float32),
            pltpu.VMEM((N_DEV, HQ, CHUNK), jnp.float32),
            pltpu.VMEM((N_DEV - 1, CHUNK, HQ, DH), jnp.float32),
            pltpu.VMEM((N_DEV - 1, HQ, CHUNK), jnp.float32),
            pltpu.SemaphoreType.DMA((N_DEV - 1,)),
            pltpu.SemaphoreType.DMA((N_DEV - 1,)),
            pltpu.SemaphoreType.DMA((N_DEV - 1,)),
            pltpu.SemaphoreType.DMA((N_DEV - 1,)),
            pltpu.SemaphoreType.DMA((N_DEV - 1,)),
            pltpu.SemaphoreType.DMA((N_DEV - 1,)),
        ],
        compiler_params=pltpu.CompilerParams(
            collective_id=0,
            vmem_limit_bytes=120 * 1024 * 1024,
        ),
    )(idx, x, Wq, K_ext, V_ext, Wo)


# device time: 79749 ns/iter; 1.3847x vs baseline; 1.1406x over previous
import os

import jax
import jax.numpy as jnp
from jax import lax
from jax.experimental import pallas as pl
from jax.experimental.pallas import tpu as pltpu

N_DEV = 16
SQ = 1024
SKV_LOCAL = 1024
HQ = 8
DH = 128
BLK = 64
CHUNK = SQ // N_DEV
NGRP = 4
GROWS = SQ // NGRP
SCALE = 0.08838834764831843
_SKIP_COMM = bool(os.environ.get("SKIP_COMM"))
_SKIP_COMPUTE = bool(os.environ.get("SKIP_COMPUTE"))

RING = [0, 1, 2, 3, 7, 6, 5, 9, 10, 11, 15, 14, 13, 12, 8, 4]
PERM = [0] * N_DEV
for _p, _d in enumerate(RING):
    PERM[_d] = _p


def _rem(v):
    return lax.rem(v + 2 * N_DEV, N_DEV)


def kernel(x, Wq, K_ext, V_ext, Wo):
    ridx = lax.axis_index("i")
    rp_ = jnp.take(jnp.array(PERM, jnp.int32), ridx)
    lneigh = jnp.take(jnp.array(RING, jnp.int32), (rp_ - 1) % N_DEV)
    rneigh = jnp.take(jnp.array(RING, jnp.int32), (rp_ + 1) % N_DEV)
    idx = jnp.stack([rp_, lneigh, rneigh]).astype(jnp.int32)

    def body(idx_ref, x_ref, wq_ref, k_ref, v_ref, wo_ref, out_ref,
             q_ref, kg_ref, vg_ref, acc_ref, l_ref, racc_ref, rl_ref,
             sL_ref, sR_ref, slL_ref, slR_ref,
             rs_s_sems, rs_r_sems, rsl_s_sems, rsl_r_sems,
             ag_s_sems, ag_r_sems):
        rp = idx_ref[0]
        left = idx_ref[1]
        right = idx_ref[2]

        xb = x_ref[0].astype(jnp.bfloat16)
        wqb = wq_ref[...].astype(jnp.bfloat16)
        q = lax.dot(xb, wqb, preferred_element_type=jnp.float32) * SCALE
        q_ref[...] = q.reshape(N_DEV, CHUNK, HQ * DH).astype(jnp.bfloat16)
        for g in range(NGRP if not _SKIP_COMPUTE else 0):
            blks = [g + NGRP * b for b in range(NGRP)]
            for h in range(HQ):
                kg_ref[g, h] = jnp.concatenate(
                    [k_ref[0, c * BLK:(c + 1) * BLK, h, :] for c in blks],
                    axis=0).astype(jnp.bfloat16)
                vg_ref[g, h] = jnp.concatenate(
                    [v_ref[0, c * BLK:(c + 1) * BLK, h, :] for c in blks],
                    axis=0).astype(jnp.bfloat16)

        def compute_pair(c):
            if _SKIP_COMPUTE:
                return
            c2 = _rem(c + 4)
            g = lax.rem(c, NGRP)
            qp = jnp.concatenate([q_ref[c], q_ref[c2]], axis=0)
            for h in range(HQ):
                qh = qp[:, h * DH:(h + 1) * DH]
                s = lax.dot_general(qh, kg_ref[g, h], (((1,), (1,)), ((), ())),
                                    preferred_element_type=jnp.float32)
                w = jnp.exp(s)
                ls = jnp.sum(w, axis=1)
                a = lax.dot(w.astype(jnp.bfloat16), vg_ref[g, h],
                            preferred_element_type=jnp.float32)
                acc_ref[c, :, h, :] = a[:CHUNK]
                acc_ref[c2, :, h, :] = a[CHUNK:]
                l_ref[c, h, :] = ls[:CHUNK]
                l_ref[c2, h, :] = ls[CHUNK:]

        compute_pair(_rem(rp - 8))
        compute_pair(_rem(rp + 7))
        compute_pair(_rem(rp - 7))
        compute_pair(_rem(rp + 6))

        barrier = pltpu.get_barrier_semaphore()
        pl.semaphore_signal(barrier, inc=1, device_id=(left,),
                            device_id_type=pl.DeviceIdType.MESH)
        pl.semaphore_signal(barrier, inc=1, device_id=(right,),
                            device_id_type=pl.DeviceIdType.MESH)
        pl.semaphore_wait(barrier, 2)

        for st in range(8):
            if not _SKIP_COMM:
                cl_s = _rem(rp - 8 + st)
                sL_ref[...] = acc_ref[cl_s].astype(jnp.bfloat16)
                slL_ref[...] = l_ref[cl_s].astype(jnp.bfloat16)
                rd_l = pltpu.make_async_remote_copy(
                    src_ref=sL_ref, dst_ref=racc_ref.at[st],
                    send_sem=rs_s_sems.at[st], recv_sem=rs_r_sems.at[st],
                    device_id=(left,), device_id_type=pl.DeviceIdType.MESH)
                rdl_l = pltpu.make_async_remote_copy(
                    src_ref=slL_ref, dst_ref=rl_ref.at[st],
                    send_sem=rsl_s_sems.at[st], recv_sem=rsl_r_sems.at[st],
                    device_id=(left,), device_id_type=pl.DeviceIdType.MESH)
                rd_l.start()
                rdl_l.start()
                if st < 7:
                    cr_s = _rem(rp + 7 - st)
                    sR_ref[...] = acc_ref[cr_s].astype(jnp.bfloat16)
                    slR_ref[...] = l_ref[cr_s].astype(jnp.bfloat16)
                    rd_r = pltpu.make_async_remote_copy(
                        src_ref=sR_ref, dst_ref=racc_ref.at[8 + st],
                        send_sem=rs_s_sems.at[8 + st],
                        recv_sem=rs_r_sems.at[8 + st],
                        device_id=(right,),
                        device_id_type=pl.DeviceIdType.MESH)
                    rdl_r = pltpu.make_async_remote_copy(
                        src_ref=slR_ref, dst_ref=rl_ref.at[8 + st],
                        send_sem=rsl_s_sems.at[8 + st],
                        recv_sem=rsl_r_sems.at[8 + st],
                        device_id=(right,),
                        device_id_type=pl.DeviceIdType.MESH)
                    rd_r.start()
                    rdl_r.start()
            if st == 0:
                compute_pair(_rem(rp + 1))
                compute_pair(rp)
            elif st == 1:
                compute_pair(_rem(rp - 1))
                compute_pair(_rem(rp - 2))
            if not _SKIP_COMM:
                rd_l.wait()
                rdl_l.wait()
                cl = _rem(rp - 7 + st)
                acc_ref[cl] = acc_ref[cl] + racc_ref[st].astype(jnp.float32)
                l_ref[cl] = l_ref[cl] + rl_ref[st].astype(jnp.float32)
                if st < 7:
                    rd_r.wait()
                    rdl_r.wait()
                    cr = _rem(rp + 6 - st)
                    acc_ref[cr] = acc_ref[cr] + racc_ref[8 + st].astype(jnp.float32)
                    l_ref[cr] = l_ref[cr] + rl_ref[8 + st].astype(jnp.float32)

        accc = acc_ref[rp]
        lc = l_ref[rp]
        parts = []
        for h in range(HQ):
            parts.append(accc[:, h, :] / lc[h][:, None])
        ctx = jnp.concatenate(parts, axis=1).astype(jnp.bfloat16)
        wob = wo_ref[...].astype(jnp.bfloat16)
        outc = lax.dot(ctx, wob,
                       preferred_element_type=jnp.float32).astype(jnp.bfloat16)
        out_ref[0, pl.ds(rp * CHUNK, CHUNK), :] = outc

        for t in range(8 if not _SKIP_COMM else 0):
            g_r = _rem(rp - t)
            ag_r = pltpu.make_async_remote_copy(
                src_ref=out_ref.at[0, pl.ds(g_r * CHUNK, CHUNK), :],
                dst_ref=out_ref.at[0, pl.ds(g_r * CHUNK, CHUNK), :],
                send_sem=ag_s_sems.at[t], recv_sem=ag_r_sems.at[t],
                device_id=(right,), device_id_type=pl.DeviceIdType.MESH)
            ag_r.start()
            if t < 7:
                g_l = _rem(rp + t)
                ag_l = pltpu.make_async_remote_copy(
                    src_ref=out_ref.at[0, pl.ds(g_l * CHUNK, CHUNK), :],
                    dst_ref=out_ref.at[0, pl.ds(g_l * CHUNK, CHUNK), :],
                    send_sem=ag_s_sems.at[8 + t], recv_sem=ag_r_sems.at[8 + t],
                    device_id=(left,), device_id_type=pl.DeviceIdType.MESH)
                ag_l.start()
            ag_r.wait()
            if t < 7:
                ag_l.wait()

    return pl.pallas_call(
        body,
        out_shape=jax.ShapeDtypeStruct((1, SQ, HQ * DH), jnp.bfloat16),
        in_specs=[pl.BlockSpec(memory_space=pltpu.SMEM)]
        + [pl.BlockSpec(memory_space=pltpu.VMEM)] * 5,
        out_specs=pl.BlockSpec(memory_space=pltpu.VMEM),
        scratch_shapes=[
            pltpu.VMEM((N_DEV, CHUNK, HQ * DH), jnp.bfloat16),
            pltpu.VMEM((NGRP, HQ, GROWS, DH), jnp.bfloat16),
            pltpu.VMEM((NGRP, HQ, GROWS, DH), jnp.bfloat16),
            pltpu.VMEM((N_DEV, CHUNK, HQ, DH), jnp.float32),
            pltpu.VMEM((N_DEV, HQ, CHUNK), jnp.float32),
            pltpu.VMEM((N_DEV - 1, CHUNK, HQ, DH), jnp.bfloat16),
            pltpu.VMEM((N_DEV - 1, HQ, CHUNK), jnp.bfloat16),
            pltpu.VMEM((CHUNK, HQ, DH), jnp.bfloat16),
            pltpu.VMEM((CHUNK, HQ, DH), jnp.bfloat16),
            pltpu.VMEM((HQ, CHUNK), jnp.bfloat16),
            pltpu.VMEM((HQ, CHUNK), jnp.bfloat16),
            pltpu.SemaphoreType.DMA((N_DEV - 1,)),
            pltpu.SemaphoreType.DMA((N_DEV - 1,)),
            pltpu.SemaphoreType.DMA((N_DEV - 1,)),
            pltpu.SemaphoreType.DMA((N_DEV - 1,)),
            pltpu.SemaphoreType.DMA((N_DEV - 1,)),
            pltpu.SemaphoreType.DMA((N_DEV - 1,)),
        ],
        compiler_params=pltpu.CompilerParams(
            collective_id=0,
            vmem_limit_bytes=120 * 1024 * 1024,
        ),
    )(idx, x, Wq, K_ext, V_ext, Wo)


# device time: 78704 ns/iter; 1.4031x vs baseline; 1.0133x over previous
import os

import jax
import jax.numpy as jnp
from jax import lax
from jax.experimental import pallas as pl
from jax.experimental.pallas import tpu as pltpu

N_DEV = 16
SQ = 1024
SKV_LOCAL = 1024
HQ = 8
DH = 128
BLK = 64
CHUNK = SQ // N_DEV
NGRP = 4
GROWS = SQ // NGRP
SCALE = 0.08838834764831843
_SKIP_COMM = bool(os.environ.get("SKIP_COMM"))
_SKIP_COMPUTE = bool(os.environ.get("SKIP_COMPUTE"))

RING = [0, 1, 2, 3, 7, 6, 5, 9, 10, 11, 15, 14, 13, 12, 8, 4]
PERM = [0] * N_DEV
for _p, _d in enumerate(RING):
    PERM[_d] = _p


def _rem(v):
    return lax.rem(v + 2 * N_DEV, N_DEV)


def kernel(x, Wq, K_ext, V_ext, Wo):
    ridx = lax.axis_index("i")
    rp_ = jnp.take(jnp.array(PERM, jnp.int32), ridx)
    lneigh = jnp.take(jnp.array(RING, jnp.int32), (rp_ - 1) % N_DEV)
    rneigh = jnp.take(jnp.array(RING, jnp.int32), (rp_ + 1) % N_DEV)
    idx = jnp.stack([rp_, lneigh, rneigh]).astype(jnp.int32)

    def body(idx_ref, x_ref, wq_ref, k_ref, v_ref, wo_ref, out_ref,
             q_ref, kg_ref, vg_ref, acc_ref, l_ref, racc_ref, rl_ref,
             sL_ref, sR_ref, slL_ref, slR_ref,
             rs_s_sems, rs_r_sems, rsl_s_sems, rsl_r_sems,
             ag_s_sems, ag_r_sems):
        rp = idx_ref[0]
        left = idx_ref[1]
        right = idx_ref[2]

        xb = x_ref[0].astype(jnp.bfloat16)
        wqb = wq_ref[...].astype(jnp.bfloat16)
        q = lax.dot(xb, wqb, preferred_element_type=jnp.float32) * SCALE
        q_ref[...] = q.reshape(N_DEV, CHUNK, HQ * DH).astype(jnp.bfloat16)
        for g in range(NGRP if not _SKIP_COMPUTE else 0):
            blks = [g + NGRP * b for b in range(NGRP)]
            for h in range(HQ):
                kg_ref[g, h] = jnp.concatenate(
                    [k_ref[0, c * BLK:(c + 1) * BLK, h, :] for c in blks],
                    axis=0).astype(jnp.bfloat16)
                vg_ref[g, h] = jnp.concatenate(
                    [v_ref[0, c * BLK:(c + 1) * BLK, h, :] for c in blks],
                    axis=0).astype(jnp.bfloat16)

        def compute_pair(c):
            if _SKIP_COMPUTE:
                return
            c2 = _rem(c + 4)
            g = lax.rem(c, NGRP)
            qp = jnp.concatenate([q_ref[c], q_ref[c2]], axis=0)
            for h in range(HQ):
                qh = qp[:, h * DH:(h + 1) * DH]
                s = lax.dot_general(qh, kg_ref[g, h], (((1,), (1,)), ((), ())),
                                    preferred_element_type=jnp.float32)
                w = jnp.exp(s)
                ls = jnp.sum(w, axis=1)
                a = lax.dot(w.astype(jnp.bfloat16), vg_ref[g, h],
                            preferred_element_type=jnp.float32)
                acc_ref[c, :, h, :] = a[:CHUNK]
                acc_ref[c2, :, h, :] = a[CHUNK:]
                l_ref[c, h, :] = ls[:CHUNK]
                l_ref[c2, h, :] = ls[CHUNK:]

        compute_pair(_rem(rp - 8))
        compute_pair(_rem(rp + 7))

        barrier = pltpu.get_barrier_semaphore()
        pl.semaphore_signal(barrier, inc=1, device_id=(left,),
                            device_id_type=pl.DeviceIdType.MESH)
        pl.semaphore_signal(barrier, inc=1, device_id=(right,),
                            device_id_type=pl.DeviceIdType.MESH)
        pl.semaphore_wait(barrier, 2)

        for st in range(8):
            if not _SKIP_COMM:
                if st == 0:
                    sL_ref[...] = acc_ref[_rem(rp - 8)].astype(jnp.bfloat16)
                    slL_ref[...] = l_ref[_rem(rp - 8)].astype(jnp.bfloat16)
                    sR_ref[...] = acc_ref[_rem(rp + 7)].astype(jnp.bfloat16)
                    slR_ref[...] = l_ref[_rem(rp + 7)].astype(jnp.bfloat16)
                rd_l = pltpu.make_async_remote_copy(
                    src_ref=sL_ref, dst_ref=racc_ref.at[st],
                    send_sem=rs_s_sems.at[st], recv_sem=rs_r_sems.at[st],
                    device_id=(left,), device_id_type=pl.DeviceIdType.MESH)
                rdl_l = pltpu.make_async_remote_copy(
                    src_ref=slL_ref, dst_ref=rl_ref.at[st],
                    send_sem=rsl_s_sems.at[st], recv_sem=rsl_r_sems.at[st],
                    device_id=(left,), device_id_type=pl.DeviceIdType.MESH)
                rd_l.start()
                rdl_l.start()
                if st < 7:
                    rd_r = pltpu.make_async_remote_copy(
                        src_ref=sR_ref, dst_ref=racc_ref.at[8 + st],
                        send_sem=rs_s_sems.at[8 + st],
                        recv_sem=rs_r_sems.at[8 + st],
                        device_id=(right,),
                        device_id_type=pl.DeviceIdType.MESH)
                    rdl_r = pltpu.make_async_remote_copy(
                        src_ref=slR_ref, dst_ref=rl_ref.at[8 + st],
                        send_sem=rsl_s_sems.at[8 + st],
                        recv_sem=rsl_r_sems.at[8 + st],
                        device_id=(right,),
                        device_id_type=pl.DeviceIdType.MESH)
                    rd_r.start()
                    rdl_r.start()
            if st == 0:
                compute_pair(_rem(rp - 7))
                compute_pair(_rem(rp + 6))
            elif st == 1:
                compute_pair(_rem(rp + 1))
                compute_pair(rp)
            elif st == 2:
                compute_pair(_rem(rp - 1))
                compute_pair(_rem(rp - 2))
            if not _SKIP_COMM:
                rd_l.wait()
                rdl_l.wait()
                cl = _rem(rp - 7 + st)
                aL = acc_ref[cl] + racc_ref[st].astype(jnp.float32)
                lL = l_ref[cl] + rl_ref[st].astype(jnp.float32)
                acc_ref[cl] = aL
                l_ref[cl] = lL
                if st < 7:
                    sL_ref[...] = aL.astype(jnp.bfloat16)
                    slL_ref[...] = lL.astype(jnp.bfloat16)
                    rd_r.wait()
                    rdl_r.wait()
                    cr = _rem(rp + 6 - st)
                    aR = acc_ref[cr] + racc_ref[8 + st].astype(jnp.float32)
                    lR = l_ref[cr] + rl_ref[8 + st].astype(jnp.float32)
                    acc_ref[cr] = aR
                    l_ref[cr] = lR
                    if st < 6:
                        sR_ref[...] = aR.astype(jnp.bfloat16)
                        slR_ref[...] = lR.astype(jnp.bfloat16)

        accc = acc_ref[rp]
        lc = l_ref[rp]
        parts = []
        for h in range(HQ):
            parts.append(accc[:, h, :] / lc[h][:, None])
        ctx = jnp.concatenate(parts, axis=1).astype(jnp.bfloat16)
        wob = wo_ref[...].astype(jnp.bfloat16)
        outc = lax.dot(ctx, wob,
                       preferred_element_type=jnp.float32).astype(jnp.bfloat16)
        out_ref[0, pl.ds(rp * CHUNK, CHUNK), :] = outc

        for t in range(8 if not _SKIP_COMM else 0):
            g_r = _rem(rp - t)
            ag_r = pltpu.make_async_remote_copy(
                src_ref=out_ref.at[0, pl.ds(g_r * CHUNK, CHUNK), :],
                dst_ref=out_ref.at[0, pl.ds(g_r * CHUNK, CHUNK), :],
                send_sem=ag_s_sems.at[t], recv_sem=ag_r_sems.at[t],
                device_id=(right,), device_id_type=pl.DeviceIdType.MESH)
            ag_r.start()
            if t < 7:
                g_l = _rem(rp + t)
                ag_l = pltpu.make_async_remote_copy(
                    src_ref=out_ref.at[0, pl.ds(g_l * CHUNK, CHUNK), :],
                    dst_ref=out_ref.at[0, pl.ds(g_l * CHUNK, CHUNK), :],
                    send_sem=ag_s_sems.at[8 + t], recv_sem=ag_r_sems.at[8 + t],
                    device_id=(left,), device_id_type=pl.DeviceIdType.MESH)
                ag_l.start()
            ag_r.wait()
            if t < 7:
                ag_l.wait()

    return pl.pallas_call(
        body,
        out_shape=jax.ShapeDtypeStruct((1, SQ, HQ * DH), jnp.bfloat16),
        in_specs=[pl.BlockSpec(memory_space=pltpu.SMEM)]
        + [pl.BlockSpec(memory_space=pltpu.VMEM)] * 5,
        out_specs=pl.BlockSpec(memory_space=pltpu.VMEM),
        scratch_shapes=[
            pltpu.VMEM((N_DEV, CHUNK, HQ * DH), jnp.bfloat16),
            pltpu.VMEM((NGRP, HQ, GROWS, DH), jnp.bfloat16),
            pltpu.VMEM((NGRP, HQ, GROWS, DH), jnp.bfloat16),
            pltpu.VMEM((N_DEV, CHUNK, HQ, DH), jnp.float32),
            pltpu.VMEM((N_DEV, HQ, CHUNK), jnp.float32),
            pltpu.VMEM((N_DEV - 1, CHUNK, HQ, DH), jnp.bfloat16),
            pltpu.VMEM((N_DEV - 1, HQ, CHUNK), jnp.bfloat16),
            pltpu.VMEM((CHUNK, HQ, DH), jnp.bfloat16),
            pltpu.VMEM((CHUNK, HQ, DH), jnp.bfloat16),
            pltpu.VMEM((HQ, CHUNK), jnp.bfloat16),
            pltpu.VMEM((HQ, CHUNK), jnp.bfloat16),
            pltpu.SemaphoreType.DMA((N_DEV - 1,)),
            pltpu.SemaphoreType.DMA((N_DEV - 1,)),
            pltpu.SemaphoreType.DMA((N_DEV - 1,)),
            pltpu.SemaphoreType.DMA((N_DEV - 1,)),
            pltpu.SemaphoreType.DMA((N_DEV - 1,)),
            pltpu.SemaphoreType.DMA((N_DEV - 1,)),
        ],
        compiler_params=pltpu.CompilerParams(
            collective_id=0,
            vmem_limit_bytes=120 * 1024 * 1024,
        ),
    )(idx, x, Wq, K_ext, V_ext, Wo)
